# baseline (speedup 1.0000x reference)
import jax
import jax.numpy as jnp
from jax import lax
from jax.experimental import pallas as pl
from jax.experimental.pallas import tpu as pltpu
from jax.experimental.pallas import tpu_sc as plsc

B, K, V = 32, 8, 100000
S = K + 1
INVALID = -1
CBLK = 12800
EPS = 1e-10
BIGI = 2**30


_CHUNKS = tuple((89600, 10400) if c == 7 else (c * CBLK, CBLK)
                for c in range(8))


NBUF = 3
NIT = B + B // 8


def _gumbel_argmax(tc, uc, off):
    col = off + lax.broadcasted_iota(jnp.int32, tc.shape, 1)
    w = EPS - jnp.log(uc + EPS)
    r = tc / w
    m = jnp.max(r, axis=1, keepdims=True)
    i = jnp.min(jnp.where(r == m, col, BIGI), axis=1, keepdims=True)
    return col, r, m, i


def _main_body(t_hbm, d_hbm, u_hbm, ids_ref,
               ia_ref, dtok_ref, ttok_ref, tbuf, dbuf, ubuf,
               tsem, dsem, usem):
    it = pl.program_id(0)

    def start(n):
        s = n % NBUF
        g8 = (n & 3) * 8
        kk = n >> 2
        pltpu.make_async_copy(t_hbm.at[pl.ds(g8, 8), kk], tbuf.at[s],
                              tsem.at[s]).start()
        pltpu.make_async_copy(u_hbm.at[pl.ds(g8, 8), kk], ubuf.at[s],
                              usem.at[s]).start()

        @pl.when(n < B)
        def _():
            pltpu.make_async_copy(d_hbm.at[pl.ds(g8, 8), kk], dbuf.at[s],
                                  dsem.at[s]).start()

    @pl.when(it == 0)
    def _():
        start(jnp.int32(0))
        start(jnp.int32(1))

    @pl.when(it + 2 < NIT)
    def _():
        start(it + 2)

    s = it % NBUF
    pltpu.make_async_copy(t_hbm.at[pl.ds(0, 8), 0], tbuf.at[s],
                          tsem.at[s]).wait()
    pltpu.make_async_copy(u_hbm.at[pl.ds(0, 8), 0], ubuf.at[s],
                          usem.at[s]).wait()

    @pl.when(it < B)
    def _():
        pltpu.make_async_copy(d_hbm.at[pl.ds(0, 8), 0], dbuf.at[s],
                              dsem.at[s]).wait()
        idv = ids_ref[it]
        bm = bi = dt = tt = None
        for c, (off, ln) in enumerate(_CHUNKS):
            tc = tbuf[s, :, pl.ds(off, ln)]
            dc = dbuf[s, :, pl.ds(off, ln)]
            uc = ubuf[s, :, pl.ds(off, ln)]
            pc = jnp.maximum(tc - dc, 1e-5)
            col, r, m, i = _gumbel_argmax(pc, uc, off)
            match = col == idv
            dsum = jnp.sum(jnp.where(match, dc, 0.0), axis=1, keepdims=True)
            tsum = jnp.sum(jnp.where(match, tc, 0.0), axis=1, keepdims=True)
            if c == 0:
                bm, bi, dt, tt = m, i, dsum, tsum
            else:
                better = m > bm
                bi = jnp.where(better, i, bi)
                bm = jnp.where(better, m, bm)
                dt = dt + dsum
                tt = tt + tsum
        ia_ref[it] = bi
        dtok_ref[it] = dt
        ttok_ref[it] = tt

    @pl.when(it >= B)
    def _():
        bm = bi = None
        for c, (off, ln) in enumerate(_CHUNKS):
            tc = tbuf[s, :, pl.ds(off, ln)]
            uc = ubuf[s, :, pl.ds(off, ln)]
            _, r, m, i = _gumbel_argmax(tc, uc, off)
            if c == 0:
                bm, bi = m, i
            else:
                better = m > bm
                bi = jnp.where(better, i, bi)
                bm = jnp.where(better, m, bm)
        ia_ref[it] = bi


def _argmax_call(draft_token_ids, draft_probs, target_probs, uniform_noise,
                 interpret=False):
    f32 = jnp.float32
    i32 = jnp.int32
    ids_prep = draft_token_ids.T.reshape(K, 4, 8).reshape(NIT - 4, 8, 1)
    ids_prep = jnp.concatenate(
        [ids_prep, jnp.zeros((4, 8, 1), i32)], axis=0)
    ia, dtok, ttok = pl.pallas_call(
        _main_body,
        grid=(NIT,),
        in_specs=[
            pl.BlockSpec(memory_space=pl.ANY),
            pl.BlockSpec(memory_space=pl.ANY),
            pl.BlockSpec(memory_space=pl.ANY),
            pl.BlockSpec((NIT, 8, 1), lambda i: (0, 0, 0)),
        ],
        out_specs=[
            pl.BlockSpec((NIT, 8, 1), lambda i: (0, 0, 0)),
            pl.BlockSpec((B, 8, 1), lambda i: (0, 0, 0)),
            pl.BlockSpec((B, 8, 1), lambda i: (0, 0, 0)),
        ],
        out_shape=[
            jax.ShapeDtypeStruct((NIT, 8, 1), i32),
            jax.ShapeDtypeStruct((B, 8, 1), f32),
            jax.ShapeDtypeStruct((B, 8, 1), f32),
        ],
        scratch_shapes=(
            [pltpu.VMEM((NBUF, 8, V), f32) for _ in range(3)]
            + [pltpu.SemaphoreType.DMA((NBUF,)) for _ in range(3)]
        ),
        compiler_params=pltpu.CompilerParams(
            dimension_semantics=("arbitrary",),
        ),
        interpret=interpret,
    )(target_probs, draft_probs, uniform_noise, ids_prep)
    rec = ia[:B].reshape(B * K)
    bon = ia[B:].reshape(B)
    dtok = dtok.reshape(B * K)
    ttok = ttok.reshape(B * K)
    return rec, bon, dtok, ttok


def _sc_body(ids_ref, us_ref, dtok_ref, ttok_ref, rec_ref, bon_ref,
             out_ref, ids_v, us_v, dtok_v, ttok_v, rec_v, bon_v,
             vals_v, out_v):
    c = lax.axis_index("c")
    s = lax.axis_index("s")

    @pl.when((c == 0) & (s == 0))
    def _():
        pltpu.sync_copy(ids_ref, ids_v)
        pltpu.sync_copy(us_ref, us_v)
        pltpu.sync_copy(dtok_ref, dtok_v)
        pltpu.sync_copy(ttok_ref, ttok_v)
        pltpu.sync_copy(rec_ref, rec_v)
        pltpu.sync_copy(bon_ref, bon_v)

        masks = [jnp.full((16,), 1, jnp.int32) for _ in range(2)]
        nas = [jnp.zeros((16,), jnp.int32) for _ in range(2)]
        for k in range(K):
            for h in range(2):
                off = k * 32 + h * 16
                u16 = us_v[pl.ds(off, 16)]
                d16 = dtok_v[pl.ds(off, 16)]
                t16 = ttok_v[pl.ds(off, 16)]
                acc = u16 <= t16 / d16
                masks[h] = jnp.where(acc, masks[h], 0)
                nas[h] = nas[h] + masks[h]
                ids16 = ids_v[pl.ds(off, 16)]
                vals_v[pl.ds(off, 16)] = jnp.where(masks[h] == 1, ids16,
                                                   INVALID)

        nexts = []
        for h in range(2):
            idxc = jnp.minimum(nas[h], K - 1)
            rec_at = jnp.zeros((16,), jnp.int32)
            for k in range(K):
                rec_k = rec_v[pl.ds(k * 32 + h * 16, 16)]
                rec_at = jnp.where(idxc == k, rec_k, rec_at)
            bon16 = bon_v[pl.ds(h * 16, 16)]
            nexts.append(jnp.where(nas[h] == K, bon16, rec_at))

        for j in range(S):
            for h in range(2):
                if j < K:
                    base = vals_v[pl.ds(j * 32 + h * 16, 16)]
                else:
                    base = jnp.full((16,), INVALID, jnp.int32)
                out_v[pl.ds(j * 32 + h * 16, 16)] = jnp.where(
                    nas[h] == j, nexts[h], base)

        pltpu.sync_copy(out_v, out_ref)


def _sc_call(ids_t, us_t, dtok, ttok, rec, bon):
    mesh = plsc.VectorSubcoreMesh(core_axis_name="c", subcore_axis_name="s")
    f32 = jnp.float32
    i32 = jnp.int32
    kern = pl.kernel(
        _sc_body,
        out_type=jax.ShapeDtypeStruct((B * S,), i32),
        mesh=mesh,
        scratch_types=[
            pltpu.VMEM((B * K,), i32),
            pltpu.VMEM((B * K,), f32),
            pltpu.VMEM((B * K,), f32),
            pltpu.VMEM((B * K,), f32),
            pltpu.VMEM((B * K,), i32),
            pltpu.VMEM((B,), i32),
            pltpu.VMEM((B * K,), i32),
            pltpu.VMEM((B * S,), i32),
        ],
    )
    return kern(ids_t, us_t, dtok, ttok, rec, bon)


def kernel(draft_token_ids, draft_probs, target_probs, uniform_samples,
           uniform_noise):
    rec, bon, dtok, ttok = _argmax_call(draft_token_ids, draft_probs,
                                        target_probs, uniform_noise)
    ids_t = draft_token_ids.T.reshape(B * K)
    us_t = uniform_samples.T.reshape(B * K)
    out = _sc_call(ids_t, us_t, dtok, ttok, rec, bon)
    return out.reshape(S, B).T


# speedup vs baseline: 2.4854x; 2.4854x over previous
import jax
import jax.numpy as jnp
from jax import lax
from jax.experimental import pallas as pl
from jax.experimental.pallas import tpu as pltpu
from jax.experimental.pallas import tpu_sc as plsc

B, K, V = 32, 8, 100000
S = K + 1
INVALID = -1
CBLK = 12800
EPS = 1e-10
BIGI = 2**30


_CHUNKS = tuple((89600, 10400) if c == 7 else (c * CBLK, CBLK)
                for c in range(8))


NBUF = 3
NIT = B + B // 8


def _gumbel_argmax(tc, uc, off):
    col = off + lax.broadcasted_iota(jnp.int32, tc.shape, 1)
    w = EPS - jnp.log(uc + EPS)
    r = tc / w
    m = jnp.max(r, axis=1, keepdims=True)
    i = jnp.min(jnp.where(r == m, col, BIGI), axis=1, keepdims=True)
    return col, r, m, i


def _main_body(t_hbm, d_hbm, u_hbm, ids_ref,
               ia_ref, dtok_ref, ttok_ref, tbuf, dbuf, ubuf,
               tsem, dsem, usem):
    it = pl.program_id(0)

    def start(n):
        s = n % NBUF
        g8 = (n & 3) * 8
        kk = n >> 2
        pltpu.make_async_copy(t_hbm.at[kk, pl.ds(g8, 8)], tbuf.at[s],
                              tsem.at[s]).start()
        pltpu.make_async_copy(u_hbm.at[kk, pl.ds(g8, 8)], ubuf.at[s],
                              usem.at[s]).start()

        @pl.when(n < B)
        def _():
            pltpu.make_async_copy(d_hbm.at[pl.ds(g8, 8), kk], dbuf.at[s],
                                  dsem.at[s]).start()

    @pl.when(it == 0)
    def _():
        start(jnp.int32(0))
        start(jnp.int32(1))

    @pl.when(it + 2 < NIT)
    def _():
        start(it + 2)

    s = it % NBUF
    pltpu.make_async_copy(t_hbm.at[0, pl.ds(0, 8)], tbuf.at[s],
                          tsem.at[s]).wait()
    pltpu.make_async_copy(u_hbm.at[0, pl.ds(0, 8)], ubuf.at[s],
                          usem.at[s]).wait()

    @pl.when(it < B)
    def _():
        pltpu.make_async_copy(d_hbm.at[pl.ds(0, 8), 0], dbuf.at[s],
                              dsem.at[s]).wait()
        idv = ids_ref[it]
        bm = bi = dt = tt = None
        for c, (off, ln) in enumerate(_CHUNKS):
            tc = tbuf[s, :, pl.ds(off, ln)]
            dc = dbuf[s, :, pl.ds(off, ln)]
            uc = ubuf[s, :, pl.ds(off, ln)]
            pc = jnp.maximum(tc - dc, 1e-5)
            col, r, m, i = _gumbel_argmax(pc, uc, off)
            match = col == idv
            dsum = jnp.sum(jnp.where(match, dc, 0.0), axis=1, keepdims=True)
            tsum = jnp.sum(jnp.where(match, tc, 0.0), axis=1, keepdims=True)
            if c == 0:
                bm, bi, dt, tt = m, i, dsum, tsum
            else:
                better = m > bm
                bi = jnp.where(better, i, bi)
                bm = jnp.where(better, m, bm)
                dt = dt + dsum
                tt = tt + tsum
        ia_ref[it] = bi
        dtok_ref[it] = dt
        ttok_ref[it] = tt

    @pl.when(it >= B)
    def _():
        bm = bi = None
        for c, (off, ln) in enumerate(_CHUNKS):
            tc = tbuf[s, :, pl.ds(off, ln)]
            uc = ubuf[s, :, pl.ds(off, ln)]
            _, r, m, i = _gumbel_argmax(tc, uc, off)
            if c == 0:
                bm, bi = m, i
            else:
                better = m > bm
                bi = jnp.where(better, i, bi)
                bm = jnp.where(better, m, bm)
        ia_ref[it] = bi


def _argmax_call(draft_token_ids, draft_probs, target_probs, uniform_noise,
                 interpret=False):
    f32 = jnp.float32
    i32 = jnp.int32
    ids_prep = draft_token_ids.T.reshape(K, 4, 8).reshape(NIT - 4, 8, 1)
    ids_prep = jnp.concatenate(
        [ids_prep, jnp.zeros((4, 8, 1), i32)], axis=0)
    ia, dtok, ttok = pl.pallas_call(
        _main_body,
        grid=(NIT,),
        in_specs=[
            pl.BlockSpec(memory_space=pl.ANY),
            pl.BlockSpec(memory_space=pl.ANY),
            pl.BlockSpec(memory_space=pl.ANY),
            pl.BlockSpec((NIT, 8, 1), lambda i: (0, 0, 0)),
        ],
        out_specs=[
            pl.BlockSpec((NIT, 8, 1), lambda i: (0, 0, 0)),
            pl.BlockSpec((B, 8, 1), lambda i: (0, 0, 0)),
            pl.BlockSpec((B, 8, 1), lambda i: (0, 0, 0)),
        ],
        out_shape=[
            jax.ShapeDtypeStruct((NIT, 8, 1), i32),
            jax.ShapeDtypeStruct((B, 8, 1), f32),
            jax.ShapeDtypeStruct((B, 8, 1), f32),
        ],
        scratch_shapes=(
            [pltpu.VMEM((NBUF, 8, V), f32) for _ in range(3)]
            + [pltpu.SemaphoreType.DMA((NBUF,)) for _ in range(3)]
        ),
        compiler_params=pltpu.CompilerParams(
            dimension_semantics=("arbitrary",),
        ),
        interpret=interpret,
    )(jnp.transpose(target_probs, (1, 0, 2)), draft_probs,
      jnp.transpose(uniform_noise, (1, 0, 2)), ids_prep)
    rec = ia[:B].reshape(B * K)
    bon = ia[B:].reshape(B)
    dtok = dtok.reshape(B * K)
    ttok = ttok.reshape(B * K)
    return rec, bon, dtok, ttok


def _sc_body(ids_ref, us_ref, dtok_ref, ttok_ref, rec_ref, bon_ref,
             out_ref, ids_v, us_v, dtok_v, ttok_v, rec_v, bon_v,
             vals_v, out_v):
    c = lax.axis_index("c")
    s = lax.axis_index("s")

    @pl.when((c == 0) & (s == 0))
    def _():
        pltpu.sync_copy(ids_ref, ids_v)
        pltpu.sync_copy(us_ref, us_v)
        pltpu.sync_copy(dtok_ref, dtok_v)
        pltpu.sync_copy(ttok_ref, ttok_v)
        pltpu.sync_copy(rec_ref, rec_v)
        pltpu.sync_copy(bon_ref, bon_v)

        masks = [jnp.full((16,), 1, jnp.int32) for _ in range(2)]
        nas = [jnp.zeros((16,), jnp.int32) for _ in range(2)]
        for k in range(K):
            for h in range(2):
                off = k * 32 + h * 16
                u16 = us_v[pl.ds(off, 16)]
                d16 = dtok_v[pl.ds(off, 16)]
                t16 = ttok_v[pl.ds(off, 16)]
                acc = u16 <= t16 / d16
                masks[h] = jnp.where(acc, masks[h], 0)
                nas[h] = nas[h] + masks[h]
                ids16 = ids_v[pl.ds(off, 16)]
                vals_v[pl.ds(off, 16)] = jnp.where(masks[h] == 1, ids16,
                                                   INVALID)

        nexts = []
        for h in range(2):
            idxc = jnp.minimum(nas[h], K - 1)
            rec_at = jnp.zeros((16,), jnp.int32)
            for k in range(K):
                rec_k = rec_v[pl.ds(k * 32 + h * 16, 16)]
                rec_at = jnp.where(idxc == k, rec_k, rec_at)
            bon16 = bon_v[pl.ds(h * 16, 16)]
            nexts.append(jnp.where(nas[h] == K, bon16, rec_at))

        for j in range(S):
            for h in range(2):
                if j < K:
                    base = vals_v[pl.ds(j * 32 + h * 16, 16)]
                else:
                    base = jnp.full((16,), INVALID, jnp.int32)
                out_v[pl.ds(j * 32 + h * 16, 16)] = jnp.where(
                    nas[h] == j, nexts[h], base)

        pltpu.sync_copy(out_v, out_ref)


def _sc_call(ids_t, us_t, dtok, ttok, rec, bon):
    mesh = plsc.VectorSubcoreMesh(core_axis_name="c", subcore_axis_name="s")
    f32 = jnp.float32
    i32 = jnp.int32
    kern = pl.kernel(
        _sc_body,
        out_type=jax.ShapeDtypeStruct((B * S,), i32),
        mesh=mesh,
        scratch_types=[
            pltpu.VMEM((B * K,), i32),
            pltpu.VMEM((B * K,), f32),
            pltpu.VMEM((B * K,), f32),
            pltpu.VMEM((B * K,), f32),
            pltpu.VMEM((B * K,), i32),
            pltpu.VMEM((B,), i32),
            pltpu.VMEM((B * K,), i32),
            pltpu.VMEM((B * S,), i32),
        ],
    )
    return kern(ids_t, us_t, dtok, ttok, rec, bon)


def kernel(draft_token_ids, draft_probs, target_probs, uniform_samples,
           uniform_noise):
    rec, bon, dtok, ttok = _argmax_call(draft_token_ids, draft_probs,
                                        target_probs, uniform_noise)
    ids_t = draft_token_ids.T.reshape(B * K)
    us_t = uniform_samples.T.reshape(B * K)
    out = _sc_call(ids_t, us_t, dtok, ttok, rec, bon)
    return out.reshape(S, B).T


# speedup vs baseline: 2.5085x; 1.0093x over previous
import jax
import jax.numpy as jnp
from jax import lax
from jax.experimental import pallas as pl
from jax.experimental.pallas import tpu as pltpu
from jax.experimental.pallas import tpu_sc as plsc

B, K, V = 32, 8, 100000
S = K + 1
INVALID = -1
CBLK = 12800
EPS = 1e-10
BIGI = 2**30


_CHUNKS = ((0, 25600), (25600, 25600), (51200, 25600), (76800, 23200))


NBUF = 3
NIT = B + B // 8


def _gumbel_argmax(tc, uc, off):
    col = off + lax.broadcasted_iota(jnp.int32, tc.shape, 1)
    w = EPS - jnp.log(uc + EPS)
    r = tc / w
    m = jnp.max(r, axis=1, keepdims=True)
    i = jnp.min(jnp.where(r == m, col, BIGI), axis=1, keepdims=True)
    return col, r, m, i


def _main_body(t_hbm, d_hbm, u_hbm, ids_ref,
               ia_ref, dtok_ref, ttok_ref, tbuf, dbuf, ubuf,
               tsem, dsem, usem):
    it = pl.program_id(0)

    def start(n):
        s = n % NBUF
        g8 = (n & 3) * 8
        kk = n >> 2
        pltpu.make_async_copy(t_hbm.at[kk, pl.ds(g8, 8)], tbuf.at[s],
                              tsem.at[s]).start()
        pltpu.make_async_copy(u_hbm.at[kk, pl.ds(g8, 8)], ubuf.at[s],
                              usem.at[s]).start()

        @pl.when(n < B)
        def _():
            pltpu.make_async_copy(d_hbm.at[pl.ds(g8, 8), kk], dbuf.at[s],
                                  dsem.at[s]).start()

    @pl.when(it == 0)
    def _():
        start(jnp.int32(0))
        start(jnp.int32(1))

    @pl.when(it + 2 < NIT)
    def _():
        start(it + 2)

    s = it % NBUF
    pltpu.make_async_copy(t_hbm.at[0, pl.ds(0, 8)], tbuf.at[s],
                          tsem.at[s]).wait()
    pltpu.make_async_copy(u_hbm.at[0, pl.ds(0, 8)], ubuf.at[s],
                          usem.at[s]).wait()

    @pl.when(it < B)
    def _():
        pltpu.make_async_copy(d_hbm.at[pl.ds(0, 8), 0], dbuf.at[s],
                              dsem.at[s]).wait()
        idv = ids_ref[it]
        bm = bi = dt = tt = None
        for c, (off, ln) in enumerate(_CHUNKS):
            tc = tbuf[s, :, pl.ds(off, ln)]
            dc = dbuf[s, :, pl.ds(off, ln)]
            uc = ubuf[s, :, pl.ds(off, ln)]
            pc = jnp.maximum(tc - dc, 1e-5)
            col, r, m, i = _gumbel_argmax(pc, uc, off)
            match = col == idv
            dsum = jnp.sum(jnp.where(match, dc, 0.0), axis=1, keepdims=True)
            tsum = jnp.sum(jnp.where(match, tc, 0.0), axis=1, keepdims=True)
            if c == 0:
                bm, bi, dt, tt = m, i, dsum, tsum
            else:
                better = m > bm
                bi = jnp.where(better, i, bi)
                bm = jnp.where(better, m, bm)
                dt = dt + dsum
                tt = tt + tsum
        ia_ref[it] = bi
        dtok_ref[it] = dt
        ttok_ref[it] = tt

    @pl.when(it >= B)
    def _():
        bm = bi = None
        for c, (off, ln) in enumerate(_CHUNKS):
            tc = tbuf[s, :, pl.ds(off, ln)]
            uc = ubuf[s, :, pl.ds(off, ln)]
            _, r, m, i = _gumbel_argmax(tc, uc, off)
            if c == 0:
                bm, bi = m, i
            else:
                better = m > bm
                bi = jnp.where(better, i, bi)
                bm = jnp.where(better, m, bm)
        ia_ref[it] = bi


def _argmax_call(draft_token_ids, draft_probs, target_probs, uniform_noise,
                 interpret=False):
    f32 = jnp.float32
    i32 = jnp.int32
    ids_prep = draft_token_ids.T.reshape(K, 4, 8).reshape(NIT - 4, 8, 1)
    ids_prep = jnp.concatenate(
        [ids_prep, jnp.zeros((4, 8, 1), i32)], axis=0)
    ia, dtok, ttok = pl.pallas_call(
        _main_body,
        grid=(NIT,),
        in_specs=[
            pl.BlockSpec(memory_space=pl.ANY),
            pl.BlockSpec(memory_space=pl.ANY),
            pl.BlockSpec(memory_space=pl.ANY),
            pl.BlockSpec((NIT, 8, 1), lambda i: (0, 0, 0)),
        ],
        out_specs=[
            pl.BlockSpec((NIT, 8, 1), lambda i: (0, 0, 0)),
            pl.BlockSpec((B, 8, 1), lambda i: (0, 0, 0)),
            pl.BlockSpec((B, 8, 1), lambda i: (0, 0, 0)),
        ],
        out_shape=[
            jax.ShapeDtypeStruct((NIT, 8, 1), i32),
            jax.ShapeDtypeStruct((B, 8, 1), f32),
            jax.ShapeDtypeStruct((B, 8, 1), f32),
        ],
        scratch_shapes=(
            [pltpu.VMEM((NBUF, 8, V), f32) for _ in range(3)]
            + [pltpu.SemaphoreType.DMA((NBUF,)) for _ in range(3)]
        ),
        compiler_params=pltpu.CompilerParams(
            dimension_semantics=("arbitrary",),
        ),
        interpret=interpret,
    )(jnp.transpose(target_probs, (1, 0, 2)), draft_probs,
      jnp.transpose(uniform_noise, (1, 0, 2)), ids_prep)
    rec = ia[:B].reshape(B * K)
    bon = ia[B:].reshape(B)
    dtok = dtok.reshape(B * K)
    ttok = ttok.reshape(B * K)
    return rec, bon, dtok, ttok


def _sc_body(ids_ref, us_ref, dtok_ref, ttok_ref, rec_ref, bon_ref,
             out_ref, ids_v, us_v, dtok_v, ttok_v, rec_v, bon_v,
             vals_v, out_v):
    c = lax.axis_index("c")
    s = lax.axis_index("s")

    @pl.when((c == 0) & (s == 0))
    def _():
        pltpu.sync_copy(ids_ref, ids_v)
        pltpu.sync_copy(us_ref, us_v)
        pltpu.sync_copy(dtok_ref, dtok_v)
        pltpu.sync_copy(ttok_ref, ttok_v)
        pltpu.sync_copy(rec_ref, rec_v)
        pltpu.sync_copy(bon_ref, bon_v)

        masks = [jnp.full((16,), 1, jnp.int32) for _ in range(2)]
        nas = [jnp.zeros((16,), jnp.int32) for _ in range(2)]
        for k in range(K):
            for h in range(2):
                off = k * 32 + h * 16
                u16 = us_v[pl.ds(off, 16)]
                d16 = dtok_v[pl.ds(off, 16)]
                t16 = ttok_v[pl.ds(off, 16)]
                acc = u16 <= t16 / d16
                masks[h] = jnp.where(acc, masks[h], 0)
                nas[h] = nas[h] + masks[h]
                ids16 = ids_v[pl.ds(off, 16)]
                vals_v[pl.ds(off, 16)] = jnp.where(masks[h] == 1, ids16,
                                                   INVALID)

        nexts = []
        for h in range(2):
            idxc = jnp.minimum(nas[h], K - 1)
            rec_at = jnp.zeros((16,), jnp.int32)
            for k in range(K):
                rec_k = rec_v[pl.ds(k * 32 + h * 16, 16)]
                rec_at = jnp.where(idxc == k, rec_k, rec_at)
            bon16 = bon_v[pl.ds(h * 16, 16)]
            nexts.append(jnp.where(nas[h] == K, bon16, rec_at))

        for j in range(S):
            for h in range(2):
                if j < K:
                    base = vals_v[pl.ds(j * 32 + h * 16, 16)]
                else:
                    base = jnp.full((16,), INVALID, jnp.int32)
                out_v[pl.ds(j * 32 + h * 16, 16)] = jnp.where(
                    nas[h] == j, nexts[h], base)

        pltpu.sync_copy(out_v, out_ref)


def _sc_call(ids_t, us_t, dtok, ttok, rec, bon):
    mesh = plsc.VectorSubcoreMesh(core_axis_name="c", subcore_axis_name="s")
    f32 = jnp.float32
    i32 = jnp.int32
    kern = pl.kernel(
        _sc_body,
        out_type=jax.ShapeDtypeStruct((B * S,), i32),
        mesh=mesh,
        scratch_types=[
            pltpu.VMEM((B * K,), i32),
            pltpu.VMEM((B * K,), f32),
            pltpu.VMEM((B * K,), f32),
            pltpu.VMEM((B * K,), f32),
            pltpu.VMEM((B * K,), i32),
            pltpu.VMEM((B,), i32),
            pltpu.VMEM((B * K,), i32),
            pltpu.VMEM((B * S,), i32),
        ],
    )
    return kern(ids_t, us_t, dtok, ttok, rec, bon)


def kernel(draft_token_ids, draft_probs, target_probs, uniform_samples,
           uniform_noise):
    rec, bon, dtok, ttok = _argmax_call(draft_token_ids, draft_probs,
                                        target_probs, uniform_noise)
    ids_t = draft_token_ids.T.reshape(B * K)
    us_t = uniform_samples.T.reshape(B * K)
    out = _sc_call(ids_t, us_t, dtok, ttok, rec, bon)
    return out.reshape(S, B).T


# speedup vs baseline: 2.5387x; 1.0121x over previous
import jax
import jax.numpy as jnp
from jax import lax
from jax.experimental import pallas as pl
from jax.experimental.pallas import tpu as pltpu
from jax.experimental.pallas import tpu_sc as plsc

B, K, V = 32, 8, 100000
S = K + 1
INVALID = -1
CBLK = 12800
EPS = 1e-10
BIGI = 2**30


_CHUNKS = ((0, 25600), (25600, 25600), (51200, 25600), (76800, 23200))


NBUF = 4
NIT = B + B // 8


def _gumbel_argmax(tc, uc, off):
    col = off + lax.broadcasted_iota(jnp.int32, tc.shape, 1)
    w = EPS - jnp.log(uc + EPS)
    r = tc / w
    m = jnp.max(r, axis=1, keepdims=True)
    i = jnp.min(jnp.where(r == m, col, BIGI), axis=1, keepdims=True)
    return col, r, m, i


def _main_body(t_hbm, d_hbm, u_hbm, ids_ref,
               ia_ref, dtok_ref, ttok_ref, tbuf, dbuf, ubuf,
               tsem, dsem, usem):
    it = pl.program_id(0)

    def start(n):
        s = n % NBUF
        g8 = (n & 3) * 8
        kk = n >> 2
        pltpu.make_async_copy(t_hbm.at[kk, pl.ds(g8, 8)], tbuf.at[s],
                              tsem.at[s]).start()
        pltpu.make_async_copy(u_hbm.at[kk, pl.ds(g8, 8)], ubuf.at[s],
                              usem.at[s]).start()

        @pl.when(n < B)
        def _():
            pltpu.make_async_copy(d_hbm.at[pl.ds(g8, 8), kk], dbuf.at[s],
                                  dsem.at[s]).start()

    @pl.when(it == 0)
    def _():
        start(jnp.int32(0))
        start(jnp.int32(1))
        start(jnp.int32(2))

    @pl.when(it + 3 < NIT)
    def _():
        start(it + 3)

    s = it % NBUF
    pltpu.make_async_copy(t_hbm.at[0, pl.ds(0, 8)], tbuf.at[s],
                          tsem.at[s]).wait()
    pltpu.make_async_copy(u_hbm.at[0, pl.ds(0, 8)], ubuf.at[s],
                          usem.at[s]).wait()

    @pl.when(it < B)
    def _():
        pltpu.make_async_copy(d_hbm.at[pl.ds(0, 8), 0], dbuf.at[s],
                              dsem.at[s]).wait()
        idv = ids_ref[it]
        bm = bi = dt = tt = None
        for c, (off, ln) in enumerate(_CHUNKS):
            tc = tbuf[s, :, pl.ds(off, ln)]
            dc = dbuf[s, :, pl.ds(off, ln)]
            uc = ubuf[s, :, pl.ds(off, ln)]
            pc = jnp.maximum(tc - dc, 1e-5)
            col, r, m, i = _gumbel_argmax(pc, uc, off)
            match = col == idv
            dsum = jnp.sum(jnp.where(match, dc, 0.0), axis=1, keepdims=True)
            tsum = jnp.sum(jnp.where(match, tc, 0.0), axis=1, keepdims=True)
            if c == 0:
                bm, bi, dt, tt = m, i, dsum, tsum
            else:
                better = m > bm
                bi = jnp.where(better, i, bi)
                bm = jnp.where(better, m, bm)
                dt = dt + dsum
                tt = tt + tsum
        ia_ref[it] = bi
        dtok_ref[it] = dt
        ttok_ref[it] = tt

    @pl.when(it >= B)
    def _():
        bm = bi = None
        for c, (off, ln) in enumerate(_CHUNKS):
            tc = tbuf[s, :, pl.ds(off, ln)]
            uc = ubuf[s, :, pl.ds(off, ln)]
            _, r, m, i = _gumbel_argmax(tc, uc, off)
            if c == 0:
                bm, bi = m, i
            else:
                better = m > bm
                bi = jnp.where(better, i, bi)
                bm = jnp.where(better, m, bm)
        ia_ref[it] = bi


def _argmax_call(draft_token_ids, draft_probs, target_probs, uniform_noise,
                 interpret=False):
    f32 = jnp.float32
    i32 = jnp.int32
    ids_prep = draft_token_ids.T.reshape(K, 4, 8).reshape(NIT - 4, 8, 1)
    ids_prep = jnp.concatenate(
        [ids_prep, jnp.zeros((4, 8, 1), i32)], axis=0)
    ia, dtok, ttok = pl.pallas_call(
        _main_body,
        grid=(NIT,),
        in_specs=[
            pl.BlockSpec(memory_space=pl.ANY),
            pl.BlockSpec(memory_space=pl.ANY),
            pl.BlockSpec(memory_space=pl.ANY),
            pl.BlockSpec((NIT, 8, 1), lambda i: (0, 0, 0)),
        ],
        out_specs=[
            pl.BlockSpec((NIT, 8, 1), lambda i: (0, 0, 0)),
            pl.BlockSpec((B, 8, 1), lambda i: (0, 0, 0)),
            pl.BlockSpec((B, 8, 1), lambda i: (0, 0, 0)),
        ],
        out_shape=[
            jax.ShapeDtypeStruct((NIT, 8, 1), i32),
            jax.ShapeDtypeStruct((B, 8, 1), f32),
            jax.ShapeDtypeStruct((B, 8, 1), f32),
        ],
        scratch_shapes=(
            [pltpu.VMEM((NBUF, 8, V), f32) for _ in range(3)]
            + [pltpu.SemaphoreType.DMA((NBUF,)) for _ in range(3)]
        ),
        compiler_params=pltpu.CompilerParams(
            dimension_semantics=("arbitrary",),
        ),
        interpret=interpret,
    )(jnp.transpose(target_probs, (1, 0, 2)), draft_probs,
      jnp.transpose(uniform_noise, (1, 0, 2)), ids_prep)
    rec = ia[:B].reshape(B * K)
    bon = ia[B:].reshape(B)
    dtok = dtok.reshape(B * K)
    ttok = ttok.reshape(B * K)
    return rec, bon, dtok, ttok


def _sc_body(ids_ref, us_ref, dtok_ref, ttok_ref, rec_ref, bon_ref,
             out_ref, ids_v, us_v, dtok_v, ttok_v, rec_v, bon_v,
             vals_v, out_v):
    c = lax.axis_index("c")
    s = lax.axis_index("s")

    @pl.when((c == 0) & (s == 0))
    def _():
        pltpu.sync_copy(ids_ref, ids_v)
        pltpu.sync_copy(us_ref, us_v)
        pltpu.sync_copy(dtok_ref, dtok_v)
        pltpu.sync_copy(ttok_ref, ttok_v)
        pltpu.sync_copy(rec_ref, rec_v)
        pltpu.sync_copy(bon_ref, bon_v)

        masks = [jnp.full((16,), 1, jnp.int32) for _ in range(2)]
        nas = [jnp.zeros((16,), jnp.int32) for _ in range(2)]
        for k in range(K):
            for h in range(2):
                off = k * 32 + h * 16
                u16 = us_v[pl.ds(off, 16)]
                d16 = dtok_v[pl.ds(off, 16)]
                t16 = ttok_v[pl.ds(off, 16)]
                acc = u16 <= t16 / d16
                masks[h] = jnp.where(acc, masks[h], 0)
                nas[h] = nas[h] + masks[h]
                ids16 = ids_v[pl.ds(off, 16)]
                vals_v[pl.ds(off, 16)] = jnp.where(masks[h] == 1, ids16,
                                                   INVALID)

        nexts = []
        for h in range(2):
            idxc = jnp.minimum(nas[h], K - 1)
            rec_at = jnp.zeros((16,), jnp.int32)
            for k in range(K):
                rec_k = rec_v[pl.ds(k * 32 + h * 16, 16)]
                rec_at = jnp.where(idxc == k, rec_k, rec_at)
            bon16 = bon_v[pl.ds(h * 16, 16)]
            nexts.append(jnp.where(nas[h] == K, bon16, rec_at))

        for j in range(S):
            for h in range(2):
                if j < K:
                    base = vals_v[pl.ds(j * 32 + h * 16, 16)]
                else:
                    base = jnp.full((16,), INVALID, jnp.int32)
                out_v[pl.ds(j * 32 + h * 16, 16)] = jnp.where(
                    nas[h] == j, nexts[h], base)

        pltpu.sync_copy(out_v, out_ref)


def _sc_call(ids_t, us_t, dtok, ttok, rec, bon):
    mesh = plsc.VectorSubcoreMesh(core_axis_name="c", subcore_axis_name="s")
    f32 = jnp.float32
    i32 = jnp.int32
    kern = pl.kernel(
        _sc_body,
        out_type=jax.ShapeDtypeStruct((B * S,), i32),
        mesh=mesh,
        scratch_types=[
            pltpu.VMEM((B * K,), i32),
            pltpu.VMEM((B * K,), f32),
            pltpu.VMEM((B * K,), f32),
            pltpu.VMEM((B * K,), f32),
            pltpu.VMEM((B * K,), i32),
            pltpu.VMEM((B,), i32),
            pltpu.VMEM((B * K,), i32),
            pltpu.VMEM((B * S,), i32),
        ],
    )
    return kern(ids_t, us_t, dtok, ttok, rec, bon)


def kernel(draft_token_ids, draft_probs, target_probs, uniform_samples,
           uniform_noise):
    rec, bon, dtok, ttok = _argmax_call(draft_token_ids, draft_probs,
                                        target_probs, uniform_noise)
    ids_t = draft_token_ids.T.reshape(B * K)
    us_t = uniform_samples.T.reshape(B * K)
    out = _sc_call(ids_t, us_t, dtok, ttok, rec, bon)
    return out.reshape(S, B).T
